# baseline (speedup 1.0000x reference)
import functools

import jax
import jax.numpy as jnp
from jax import lax
from jax.experimental import pallas as pl
from jax.experimental.pallas import tpu as pltpu
from jax.experimental.pallas import tpu_sc as plsc

N_NODES = 10000
HIDDEN = 128
MSG = 128
N_TYPES = 3
E_PER_TYPE = 106667
SMALL_NUMBER = 1e-07

N_EDGES = N_TYPES * E_PER_TYPE
NW = 32
KB = 32
EDGES_P = 323584
SCAN_BLK = 1024
N_SCAN = EDGES_P // SCAN_BLK
OWN = 320
OWN_ROWS = NW * OWN
WL_CAP = 11264


_ROW_BLK = 1000


def _ab_body(x_ref, w_ref, b_ref, a_ref, bt_ref):
    xb = x_ref[...]
    w = w_ref[0]
    a_ref[0] = jnp.dot(xb, w[:HIDDEN], preferred_element_type=jnp.float32) + b_ref[0]
    bt_ref[0, 0] = jnp.dot(xb, w[HIDDEN:], preferred_element_type=jnp.float32)


TSTRIDE = 16384


def _compute_ab(x, W, b):
    grid = (N_TYPES, 3, N_NODES // _ROW_BLK)
    out_shape = [
        jax.ShapeDtypeStruct((3, N_TYPES * N_NODES, MSG), jnp.float32),
        jax.ShapeDtypeStruct((3, N_TYPES, TSTRIDE, MSG), jnp.float32),
    ]
    nb = N_NODES // _ROW_BLK
    return pl.pallas_call(
        _ab_body,
        grid=grid,
        in_specs=[
            pl.BlockSpec((_ROW_BLK, HIDDEN), lambda t, c, i: (i, 0)),
            pl.BlockSpec((1, 2 * HIDDEN, MSG), lambda t, c, i: (t, 0, c)),
            pl.BlockSpec((1, 1, MSG), lambda t, c, i: (t, 0, c)),
        ],
        out_specs=[
            pl.BlockSpec((1, _ROW_BLK, MSG), lambda t, c, i: (c, t * nb + i, 0)),
            pl.BlockSpec((1, 1, _ROW_BLK, MSG), lambda t, c, i: (c, t, i, 0)),
        ],
        out_shape=out_shape,
    )(x, W, b[:, None, :])


def _zero_acc(ref, rows):
    z = jnp.zeros((16,), jnp.float32)

    def body(i, _):
        for j in range(MSG // 16):
            ref[i, pl.ds(j * 16, 16)] = z
        return 0

    lax.fori_loop(0, rows, body, 0)


@functools.partial(
    pl.kernel,
    out_type=[
        jax.ShapeDtypeStruct((OWN_ROWS, MSG), jnp.float32),
        jax.ShapeDtypeStruct((OWN_ROWS, MSG), jnp.float32),
        jax.ShapeDtypeStruct((OWN_ROWS, MSG), jnp.float32),
        jax.ShapeDtypeStruct((OWN_ROWS, MSG), jnp.float32),
    ],
    mesh=plsc.VectorSubcoreMesh(core_axis_name="c", subcore_axis_name="s"),
    compiler_params=pltpu.CompilerParams(needs_layout_passes=False),
    scratch_types=[
        pltpu.VMEM((WL_CAP,), jnp.int32),
        pltpu.VMEM((WL_CAP,), jnp.int32),
        pltpu.VMEM((SCAN_BLK,), jnp.int32),
        pltpu.VMEM((SCAN_BLK,), jnp.int32),
        pltpu.VMEM((SCAN_BLK,), jnp.int32),
        pltpu.VMEM((SCAN_BLK,), jnp.int32),
        pltpu.VMEM((SCAN_BLK,), jnp.int32),
        pltpu.VMEM((SCAN_BLK,), jnp.int32),
        pltpu.VMEM((KB, MSG), jnp.float32),
        pltpu.VMEM((KB, MSG), jnp.float32),
        pltpu.VMEM((KB, MSG), jnp.float32),
        pltpu.VMEM((KB, MSG), jnp.float32),
        pltpu.VMEM((OWN, MSG), jnp.float32),
        pltpu.VMEM((OWN, MSG), jnp.float32),
        pltpu.VMEM((OWN,), jnp.float32),
        pltpu.SemaphoreType.DMA,
        pltpu.SemaphoreType.DMA,
        pltpu.SemaphoreType.DMA,
        pltpu.SemaphoreType.DMA,
        pltpu.SemaphoreType.DMA,
        pltpu.SemaphoreType.DMA,
    ],
)
def _sc_aggr(a0, b0, a1, b1, a2, b2, sadj, tadj, traw,
             sum_out, mean_out, std_out, max_out,
             wl_s, wl_t, sstg0, sstg1, tstg0, tstg1, rstg0, rstg1,
             ar0, ar1, br0, br1,
             acc1, acc2, cnt, semA0, semA1, semB0, semB1, semC0, semC1):
    c = lax.axis_index("c")
    s = lax.axis_index("s")
    wid = c * 16 + s
    lo = wid * OWN

    iota = lax.iota(jnp.int32, 16)
    zero16i = jnp.zeros((16,), jnp.int32)
    zero16 = jnp.zeros((16,), jnp.float32)
    eps16 = jnp.full((16,), SMALL_NUMBER, jnp.float32)

    def wlz(i, _):
        wl_s[pl.ds(i * 16, 16)] = zero16i
        wl_t[pl.ds(i * 16, 16)] = zero16i
        return 0

    lax.fori_loop(0, WL_CAP // 16, wlz, 0)

    stg = ((sstg0, tstg0, rstg0, semA0, semB0, semC0),
           (sstg1, tstg1, rstg1, semA1, semB1, semC1))

    def scan_issue(bi, par):
        sb, tb, rb, sa, sb_, sc_ = stg[par]
        base = jnp.minimum(bi, N_SCAN - 1) * SCAN_BLK
        pltpu.async_copy(sadj.at[pl.ds(base, SCAN_BLK)], sb, sa)
        pltpu.async_copy(tadj.at[pl.ds(base, SCAN_BLK)], tb, sb_)
        pltpu.async_copy(traw.at[pl.ds(base, SCAN_BLK)], rb, sc_)

    def scan_wait(par):
        sb, tb, rb, sa, sb_, sc_ = stg[par]
        pltpu.make_async_copy(sadj.at[pl.ds(0, SCAN_BLK)], sb, sa).wait()
        pltpu.make_async_copy(tadj.at[pl.ds(0, SCAN_BLK)], tb, sb_).wait()
        pltpu.make_async_copy(traw.at[pl.ds(0, SCAN_BLK)], rb, sc_).wait()

    def scan_body(par, bi, n):
        sb, tb, rb = stg[par][0], stg[par][1], stg[par][2]
        scan_wait(par)

        def grp(g, n):
            sl = pl.ds(g * 16, 16)
            tr = rb[sl]
            m = (tr >= lo) & (tr < lo + OWN)
            nc = jnp.minimum(n, WL_CAP - 64)
            mv = m.astype(jnp.int32)
            pos = plsc.cumsum(mv) + jnp.broadcast_to(nc - 1, (16,))
            plsc.store_scatter(wl_s, [pos], sb[sl], mask=m)
            plsc.store_scatter(wl_t, [pos], tb[sl], mask=m)
            return n + jnp.sum(mv)

        n = plsc.parallel_loop(0, SCAN_BLK // 16, 1, unroll=2, carry=n)(
            lambda g, nn: grp(g, nn))
        scan_issue(bi + 2, par)
        return n

    scan_issue(0, 0)
    scan_issue(1, 1)

    def scan_pair(pi, n):
        n = scan_body(0, pi * 2, n)
        n = scan_body(1, pi * 2 + 1, n)
        return n

    cnt_e = lax.fori_loop(0, N_SCAN // 2, scan_pair, jnp.int32(0))
    scan_wait(0)
    scan_wait(1)

    nblocks = (cnt_e + KB - 1) // KB
    npair = (nblocks + 2) // 2

    def sweep(atab, btab, update, par_safe, update2=None):
        def issue(b, ar, br, sa, sb_):
            base = b * KB
            pltpu.async_copy(atab.at[wl_s.at[pl.ds(base, KB)]], ar, sa)
            pltpu.async_copy(btab.at[wl_t.at[pl.ds(base, KB)]], br, sb_)

        def wait(ar, br, sa, sb_):
            pltpu.make_async_copy(atab.at[wl_s.at[pl.ds(0, KB)]], ar, sa).wait()
            pltpu.make_async_copy(btab.at[wl_t.at[pl.ds(0, KB)]], br, sb_).wait()

        def proc(b, ar, br, sa, sb_):
            wait(ar, br, sa, sb_)

            def edge_body(e, upd):
                i = b * KB + e
                valid = jnp.broadcast_to(i < cnt_e, (16,))
                tv = plsc.load_gather(wl_t, [jnp.broadcast_to(i, (16,))])
                tloc = jnp.clip((tv & (TSTRIDE - 1)) - lo, 0, OWN - 1)
                upd(ar, br, e, tloc, valid)

            if par_safe:
                @plsc.parallel_loop(0, KB, 1, unroll=4)
                def _(e):
                    edge_body(e, update)
            else:
                def edge(e2, _):
                    edge_body(2 * e2, update)
                    edge_body(2 * e2 + 1, update2)
                    return 0

                lax.fori_loop(0, KB // 2, edge, 0)
            issue(b + 2, ar, br, sa, sb_)

        issue(0, ar0, br0, semA0, semB0)
        issue(1, ar1, br1, semA1, semB1)

        def pair(pi, _):
            proc(pi * 2, ar0, br0, semA0, semB0)
            proc(pi * 2 + 1, ar1, br1, semA1, semB1)
            return 0

        lax.fori_loop(0, npair, pair, 0)
        wait(ar0, br0, semA0, semB0)
        wait(ar1, br1, semA1, semB1)

    def upd_sum(ar, br, e, tloc, valid):
        for j in range(MSG // 16):
            sl = pl.ds(j * 16, 16)
            vals = jnp.maximum(ar[e, sl] + br[e, sl], zero16)
            cols = iota + (j * 16)
            plsc.addupdate_scatter(acc1, [tloc, cols], vals, mask=valid)

    def upd_mean(ar, br, e, tloc, valid):
        upd_sum(ar, br, e, tloc, valid)
        vm = valid & (iota == 0)
        plsc.addupdate_scatter(cnt, [tloc], jnp.ones((16,), jnp.float32),
                               mask=vm)

    def upd_std(ar, br, e, tloc, valid):
        for j in range(MSG // 16):
            sl = pl.ds(j * 16, 16)
            m = jnp.maximum(ar[e, sl] + br[e, sl], zero16)
            cols = iota + (j * 16)
            mu = plsc.load_gather(acc1, [tloc, cols])
            pe = jnp.maximum(m * m - mu * mu, zero16) + eps16
            plsc.addupdate_scatter(acc2, [tloc, cols], pe, mask=valid)

    def _max_into(acc, ar, br, e, tloc, valid):
        news = []
        for j in range(MSG // 16):
            sl = pl.ds(j * 16, 16)
            vals = jnp.maximum(ar[e, sl] + br[e, sl], zero16)
            cols = iota + (j * 16)
            old = plsc.load_gather(acc, [tloc, cols])
            news.append(jnp.maximum(old, vals))
        for j in range(MSG // 16):
            cols = iota + (j * 16)
            plsc.store_scatter(acc, [tloc, cols], news[j], mask=valid)

    def upd_max(ar, br, e, tloc, valid):
        _max_into(acc1, ar, br, e, tloc, valid)

    def upd_max2(ar, br, e, tloc, valid):
        _max_into(acc2, ar, br, e, tloc, valid)

    _zero_acc(acc1, OWN)
    sweep(a0, b0, upd_sum, True)
    pltpu.sync_copy(acc1, sum_out.at[pl.ds(lo, OWN)])

    _zero_acc(acc1, OWN)

    def cz(i, _):
        cnt[pl.ds(i * 16, 16)] = zero16
        return 0

    lax.fori_loop(0, OWN // 16, cz, 0)
    sweep(a1, b1, upd_mean, True)

    def div_row(i, _):
        d = jnp.maximum(plsc.load_gather(cnt, [jnp.broadcast_to(i, (16,))]), 1.0)
        for j in range(MSG // 16):
            sl = pl.ds(j * 16, 16)
            acc1[i, sl] = acc1[i, sl] / d
        return 0

    lax.fori_loop(0, OWN, div_row, 0)
    pltpu.sync_copy(acc1, mean_out.at[pl.ds(lo, OWN)])

    _zero_acc(acc2, OWN)
    sweep(a1, b1, upd_std, True)
    pltpu.sync_copy(acc2, std_out.at[pl.ds(lo, OWN)])

    _zero_acc(acc1, OWN)
    _zero_acc(acc2, OWN)
    sweep(a2, b2, upd_max, False, upd_max2)

    def mmerge(i, _):
        for j in range(MSG // 16):
            sl = pl.ds(j * 16, 16)
            acc1[i, sl] = jnp.maximum(acc1[i, sl], acc2[i, sl])
        return 0

    lax.fori_loop(0, OWN, mmerge, 0)
    pltpu.sync_copy(acc1, max_out.at[pl.ds(lo, OWN)])


def _assemble_body(sum_ref, mean_ref, stp_ref, mx_ref, out_ref):
    std_agg = jnp.sqrt(stp_ref[...])
    out_ref[...] = jnp.concatenate(
        [sum_ref[...], mean_ref[...], std_agg, mx_ref[...]], axis=1)


def _assemble(sum_agg, mean_agg, std_sums, max_agg):
    blk = 1000
    grid = (N_NODES // blk,)
    return pl.pallas_call(
        _assemble_body,
        grid=grid,
        in_specs=[pl.BlockSpec((blk, MSG), lambda i: (i, 0))] * 4,
        out_specs=pl.BlockSpec((blk, 4 * MSG), lambda i: (i, 0)),
        out_shape=jax.ShapeDtypeStruct((N_NODES, 4 * MSG), jnp.float32),
    )(sum_agg, mean_agg, std_sums, max_agg)


def kernel(x, adj_lists, W, b):
    A, B = _compute_ab(x, W, b)

    srcs = adj_lists[:, :, 0]
    tgts = adj_lists[:, :, 1]
    offs = (jnp.arange(N_TYPES, dtype=jnp.int32) * N_NODES)[:, None]
    offs_t = (jnp.arange(N_TYPES, dtype=jnp.int32) * TSTRIDE)[:, None]
    src_adj = (srcs + offs).reshape(-1)
    tgt_adj = (tgts + offs_t).reshape(-1)
    tgt_raw = tgts.reshape(-1)
    pad = EDGES_P - N_EDGES
    src_adj = jnp.concatenate([src_adj, jnp.zeros((pad,), jnp.int32)])
    tgt_adj = jnp.concatenate([tgt_adj, jnp.zeros((pad,), jnp.int32)])
    tgt_raw = jnp.concatenate([tgt_raw, jnp.full((pad,), 1 << 20, jnp.int32)])

    Bf = B.reshape(3, N_TYPES * TSTRIDE, MSG)
    sum_agg, mean_agg, std_sums, max_agg = _sc_aggr(
        A[0], Bf[0], A[1], Bf[1], A[2], Bf[2], src_adj, tgt_adj, tgt_raw)

    return _assemble(sum_agg[:N_NODES], mean_agg[:N_NODES],
                     std_sums[:N_NODES], max_agg[:N_NODES])


# speedup vs baseline: 1.0048x; 1.0048x over previous
import functools

import jax
import jax.numpy as jnp
from jax import lax
from jax.experimental import pallas as pl
from jax.experimental.pallas import tpu as pltpu
from jax.experimental.pallas import tpu_sc as plsc

N_NODES = 10000
HIDDEN = 128
MSG = 128
N_TYPES = 3
E_PER_TYPE = 106667
SMALL_NUMBER = 1e-07

N_EDGES = N_TYPES * E_PER_TYPE
NW = 32
KB = 32
EDGES_P = 323584
SCAN_BLK = 1024
N_SCAN = EDGES_P // SCAN_BLK
OWN = 320
OWN_ROWS = NW * OWN
WL_CAP = 11264


_ROW_BLK = 1000


def _ab_body(x_ref, w_ref, b_ref, a_ref, bt_ref):
    xb = x_ref[...]
    w = w_ref[0]
    a_ref[0] = jnp.dot(xb, w[:HIDDEN], preferred_element_type=jnp.float32) + b_ref[0]
    bt_ref[0, 0] = jnp.dot(xb, w[HIDDEN:], preferred_element_type=jnp.float32)


TSTRIDE = 16384


def _compute_ab(x, W, b):
    grid = (N_TYPES, 3, N_NODES // _ROW_BLK)
    out_shape = [
        jax.ShapeDtypeStruct((3, N_TYPES * N_NODES, MSG), jnp.float32),
        jax.ShapeDtypeStruct((3, N_TYPES, TSTRIDE, MSG), jnp.float32),
    ]
    nb = N_NODES // _ROW_BLK
    return pl.pallas_call(
        _ab_body,
        grid=grid,
        in_specs=[
            pl.BlockSpec((_ROW_BLK, HIDDEN), lambda t, c, i: (i, 0)),
            pl.BlockSpec((1, 2 * HIDDEN, MSG), lambda t, c, i: (t, 0, c)),
            pl.BlockSpec((1, 1, MSG), lambda t, c, i: (t, 0, c)),
        ],
        out_specs=[
            pl.BlockSpec((1, _ROW_BLK, MSG), lambda t, c, i: (c, t * nb + i, 0)),
            pl.BlockSpec((1, 1, _ROW_BLK, MSG), lambda t, c, i: (c, t, i, 0)),
        ],
        out_shape=out_shape,
    )(x, W, b[:, None, :])


def _zero_acc(ref, rows):
    z = jnp.zeros((16,), jnp.float32)

    def body(i, _):
        for j in range(MSG // 16):
            ref[i, pl.ds(j * 16, 16)] = z
        return 0

    lax.fori_loop(0, rows, body, 0)


@functools.partial(
    pl.kernel,
    out_type=[
        jax.ShapeDtypeStruct((OWN_ROWS, MSG), jnp.float32),
        jax.ShapeDtypeStruct((OWN_ROWS, MSG), jnp.float32),
        jax.ShapeDtypeStruct((OWN_ROWS, MSG), jnp.float32),
        jax.ShapeDtypeStruct((OWN_ROWS, MSG), jnp.float32),
    ],
    mesh=plsc.VectorSubcoreMesh(core_axis_name="c", subcore_axis_name="s"),
    compiler_params=pltpu.CompilerParams(needs_layout_passes=False),
    scratch_types=[
        pltpu.VMEM((WL_CAP,), jnp.int32),
        pltpu.VMEM((WL_CAP,), jnp.int32),
        pltpu.VMEM((SCAN_BLK,), jnp.int32),
        pltpu.VMEM((SCAN_BLK,), jnp.int32),
        pltpu.VMEM((SCAN_BLK,), jnp.int32),
        pltpu.VMEM((SCAN_BLK,), jnp.int32),
        pltpu.VMEM((SCAN_BLK,), jnp.int32),
        pltpu.VMEM((SCAN_BLK,), jnp.int32),
        pltpu.VMEM((KB, MSG), jnp.float32),
        pltpu.VMEM((KB, MSG), jnp.float32),
        pltpu.VMEM((KB, MSG), jnp.float32),
        pltpu.VMEM((KB, MSG), jnp.float32),
        pltpu.VMEM((OWN, MSG), jnp.float32),
        pltpu.VMEM((OWN, MSG), jnp.float32),
        pltpu.VMEM((OWN,), jnp.float32),
        pltpu.SemaphoreType.DMA,
        pltpu.SemaphoreType.DMA,
        pltpu.SemaphoreType.DMA,
        pltpu.SemaphoreType.DMA,
        pltpu.SemaphoreType.DMA,
        pltpu.SemaphoreType.DMA,
    ],
)
def _sc_aggr(a0, b0, a1, b1, a2, b2, sadj, tadj, traw,
             sum_out, mean_out, std_out, max_out,
             wl_s, wl_t, sstg0, sstg1, tstg0, tstg1, rstg0, rstg1,
             ar0, ar1, br0, br1,
             acc1, acc2, cnt, semA0, semA1, semB0, semB1, semC0, semC1):
    c = lax.axis_index("c")
    s = lax.axis_index("s")
    wid = c * 16 + s
    lo = wid * OWN

    iota = lax.iota(jnp.int32, 16)
    zero16i = jnp.zeros((16,), jnp.int32)
    zero16 = jnp.zeros((16,), jnp.float32)
    eps16 = jnp.full((16,), SMALL_NUMBER, jnp.float32)

    def wlz(i, _):
        wl_s[pl.ds(i * 16, 16)] = zero16i
        wl_t[pl.ds(i * 16, 16)] = zero16i
        return 0

    lax.fori_loop(0, WL_CAP // 16, wlz, 0)

    stg = ((sstg0, tstg0, rstg0, semA0, semB0, semC0),
           (sstg1, tstg1, rstg1, semA1, semB1, semC1))

    def scan_issue(bi, par):
        sb, tb, rb, sa, sb_, sc_ = stg[par]
        base = jnp.minimum(bi, N_SCAN - 1) * SCAN_BLK
        pltpu.async_copy(sadj.at[pl.ds(base, SCAN_BLK)], sb, sa)
        pltpu.async_copy(tadj.at[pl.ds(base, SCAN_BLK)], tb, sb_)
        pltpu.async_copy(traw.at[pl.ds(base, SCAN_BLK)], rb, sc_)

    def scan_wait(par):
        sb, tb, rb, sa, sb_, sc_ = stg[par]
        pltpu.make_async_copy(sadj.at[pl.ds(0, SCAN_BLK)], sb, sa).wait()
        pltpu.make_async_copy(tadj.at[pl.ds(0, SCAN_BLK)], tb, sb_).wait()
        pltpu.make_async_copy(traw.at[pl.ds(0, SCAN_BLK)], rb, sc_).wait()

    def scan_body(par, bi, n):
        sb, tb, rb = stg[par][0], stg[par][1], stg[par][2]
        scan_wait(par)

        def grp(g, n):
            sl = pl.ds(g * 16, 16)
            tr = rb[sl]
            m = (tr >= lo) & (tr < lo + OWN)
            nc = jnp.minimum(n, WL_CAP - 64)
            mv = m.astype(jnp.int32)
            pos = plsc.cumsum(mv) + jnp.broadcast_to(nc - 1, (16,))
            plsc.store_scatter(wl_s, [pos], sb[sl], mask=m)
            plsc.store_scatter(wl_t, [pos], tb[sl], mask=m)
            return n + jnp.sum(mv)

        n = plsc.parallel_loop(0, SCAN_BLK // 16, 1, unroll=2, carry=n)(
            lambda g, nn: grp(g, nn))
        scan_issue(bi + 2, par)
        return n

    scan_issue(0, 0)
    scan_issue(1, 1)

    def scan_pair(pi, n):
        n = scan_body(0, pi * 2, n)
        n = scan_body(1, pi * 2 + 1, n)
        return n

    cnt_e = lax.fori_loop(0, N_SCAN // 2, scan_pair, jnp.int32(0))
    scan_wait(0)
    scan_wait(1)

    nblocks = (cnt_e + KB - 1) // KB
    npair = (nblocks + 2) // 2

    def sweep(atab, btab, update, par_safe, update2=None):
        def issue(b, ar, br, sa, sb_):
            base = b * KB
            pltpu.async_copy(atab.at[wl_s.at[pl.ds(base, KB)]], ar, sa)
            pltpu.async_copy(btab.at[wl_t.at[pl.ds(base, KB)]], br, sb_)

        def wait(ar, br, sa, sb_):
            pltpu.make_async_copy(atab.at[wl_s.at[pl.ds(0, KB)]], ar, sa).wait()
            pltpu.make_async_copy(btab.at[wl_t.at[pl.ds(0, KB)]], br, sb_).wait()

        def proc(b, ar, br, sa, sb_):
            wait(ar, br, sa, sb_)

            def edge_body(e, upd):
                i = b * KB + e
                valid = jnp.broadcast_to(i < cnt_e, (16,))
                tv = plsc.load_gather(wl_t, [jnp.broadcast_to(i, (16,))])
                tloc = jnp.clip((tv & (TSTRIDE - 1)) - lo, 0, OWN - 1)
                upd(ar, br, e, tloc, valid)

            if par_safe:
                @plsc.parallel_loop(0, KB, 1, unroll=2)
                def _(e):
                    edge_body(e, update)
            else:
                def edge(e2, _):
                    edge_body(2 * e2, update)
                    edge_body(2 * e2 + 1, update2)
                    return 0

                lax.fori_loop(0, KB // 2, edge, 0)
            issue(b + 2, ar, br, sa, sb_)

        issue(0, ar0, br0, semA0, semB0)
        issue(1, ar1, br1, semA1, semB1)

        def pair(pi, _):
            proc(pi * 2, ar0, br0, semA0, semB0)
            proc(pi * 2 + 1, ar1, br1, semA1, semB1)
            return 0

        lax.fori_loop(0, npair, pair, 0)
        wait(ar0, br0, semA0, semB0)
        wait(ar1, br1, semA1, semB1)

    def upd_sum(ar, br, e, tloc, valid):
        for j in range(MSG // 16):
            sl = pl.ds(j * 16, 16)
            vals = jnp.maximum(ar[e, sl] + br[e, sl], zero16)
            cols = iota + (j * 16)
            plsc.addupdate_scatter(acc1, [tloc, cols], vals, mask=valid)

    def upd_mean(ar, br, e, tloc, valid):
        upd_sum(ar, br, e, tloc, valid)
        vm = valid & (iota == 0)
        plsc.addupdate_scatter(cnt, [tloc], jnp.ones((16,), jnp.float32),
                               mask=vm)

    def upd_std(ar, br, e, tloc, valid):
        for j in range(MSG // 16):
            sl = pl.ds(j * 16, 16)
            m = jnp.maximum(ar[e, sl] + br[e, sl], zero16)
            cols = iota + (j * 16)
            mu = plsc.load_gather(acc1, [tloc, cols])
            pe = jnp.maximum(m * m - mu * mu, zero16) + eps16
            plsc.addupdate_scatter(acc2, [tloc, cols], pe, mask=valid)

    def _max_into(acc, ar, br, e, tloc, valid):
        news = []
        for j in range(MSG // 16):
            sl = pl.ds(j * 16, 16)
            vals = jnp.maximum(ar[e, sl] + br[e, sl], zero16)
            cols = iota + (j * 16)
            old = plsc.load_gather(acc, [tloc, cols])
            news.append(jnp.maximum(old, vals))
        for j in range(MSG // 16):
            cols = iota + (j * 16)
            plsc.store_scatter(acc, [tloc, cols], news[j], mask=valid)

    def upd_max(ar, br, e, tloc, valid):
        _max_into(acc1, ar, br, e, tloc, valid)

    def upd_max2(ar, br, e, tloc, valid):
        _max_into(acc2, ar, br, e, tloc, valid)

    _zero_acc(acc1, OWN)
    sweep(a0, b0, upd_sum, True)
    pltpu.sync_copy(acc1, sum_out.at[pl.ds(lo, OWN)])

    _zero_acc(acc1, OWN)

    def cz(i, _):
        cnt[pl.ds(i * 16, 16)] = zero16
        return 0

    lax.fori_loop(0, OWN // 16, cz, 0)
    sweep(a1, b1, upd_mean, True)

    def div_row(i, _):
        d = jnp.maximum(plsc.load_gather(cnt, [jnp.broadcast_to(i, (16,))]), 1.0)
        for j in range(MSG // 16):
            sl = pl.ds(j * 16, 16)
            acc1[i, sl] = acc1[i, sl] / d
        return 0

    lax.fori_loop(0, OWN, div_row, 0)
    pltpu.sync_copy(acc1, mean_out.at[pl.ds(lo, OWN)])

    _zero_acc(acc2, OWN)
    sweep(a1, b1, upd_std, True)
    pltpu.sync_copy(acc2, std_out.at[pl.ds(lo, OWN)])

    _zero_acc(acc1, OWN)
    _zero_acc(acc2, OWN)
    sweep(a2, b2, upd_max, False, upd_max2)

    def mmerge(i, _):
        for j in range(MSG // 16):
            sl = pl.ds(j * 16, 16)
            acc1[i, sl] = jnp.maximum(acc1[i, sl], acc2[i, sl])
        return 0

    lax.fori_loop(0, OWN, mmerge, 0)
    pltpu.sync_copy(acc1, max_out.at[pl.ds(lo, OWN)])


def _assemble_body(sum_ref, mean_ref, stp_ref, mx_ref, out_ref):
    std_agg = jnp.sqrt(stp_ref[...])
    out_ref[...] = jnp.concatenate(
        [sum_ref[...], mean_ref[...], std_agg, mx_ref[...]], axis=1)


def _assemble(sum_agg, mean_agg, std_sums, max_agg):
    blk = 1000
    grid = (N_NODES // blk,)
    return pl.pallas_call(
        _assemble_body,
        grid=grid,
        in_specs=[pl.BlockSpec((blk, MSG), lambda i: (i, 0))] * 4,
        out_specs=pl.BlockSpec((blk, 4 * MSG), lambda i: (i, 0)),
        out_shape=jax.ShapeDtypeStruct((N_NODES, 4 * MSG), jnp.float32),
    )(sum_agg, mean_agg, std_sums, max_agg)


def kernel(x, adj_lists, W, b):
    A, B = _compute_ab(x, W, b)

    srcs = adj_lists[:, :, 0]
    tgts = adj_lists[:, :, 1]
    offs = (jnp.arange(N_TYPES, dtype=jnp.int32) * N_NODES)[:, None]
    offs_t = (jnp.arange(N_TYPES, dtype=jnp.int32) * TSTRIDE)[:, None]
    src_adj = (srcs + offs).reshape(-1)
    tgt_adj = (tgts + offs_t).reshape(-1)
    tgt_raw = tgts.reshape(-1)
    pad = EDGES_P - N_EDGES
    src_adj = jnp.concatenate([src_adj, jnp.zeros((pad,), jnp.int32)])
    tgt_adj = jnp.concatenate([tgt_adj, jnp.zeros((pad,), jnp.int32)])
    tgt_raw = jnp.concatenate([tgt_raw, jnp.full((pad,), 1 << 20, jnp.int32)])

    Bf = B.reshape(3, N_TYPES * TSTRIDE, MSG)
    sum_agg, mean_agg, std_sums, max_agg = _sc_aggr(
        A[0], Bf[0], A[1], Bf[1], A[2], Bf[2], src_adj, tgt_adj, tgt_raw)

    return _assemble(sum_agg[:N_NODES], mean_agg[:N_NODES],
                     std_sums[:N_NODES], max_agg[:N_NODES])


# speedup vs baseline: 1.0659x; 1.0609x over previous
import functools

import jax
import jax.numpy as jnp
from jax import lax
from jax.experimental import pallas as pl
from jax.experimental.pallas import tpu as pltpu
from jax.experimental.pallas import tpu_sc as plsc

N_NODES = 10000
HIDDEN = 128
MSG = 128
N_TYPES = 3
E_PER_TYPE = 106667
SMALL_NUMBER = 1e-07

N_EDGES = N_TYPES * E_PER_TYPE
NW = 32
KB = 16
EDGES_P = 323584
SCAN_BLK = 1024
N_SCAN = EDGES_P // SCAN_BLK
OWN = 320
OWN_ROWS = NW * OWN
WL_CAP = 11264


_ROW_BLK = 1000


def _ab_body(x_ref, w_ref, b_ref, a_ref, bt_ref):
    xb = x_ref[...]
    w = w_ref[0]
    a_ref[0] = jnp.dot(xb, w[:HIDDEN], preferred_element_type=jnp.float32) + b_ref[0]
    bt_ref[0, 0] = jnp.dot(xb, w[HIDDEN:], preferred_element_type=jnp.float32)


TSTRIDE = 16384


def _compute_ab(x, W, b):
    grid = (N_TYPES, 3, N_NODES // _ROW_BLK)
    out_shape = [
        jax.ShapeDtypeStruct((3, N_TYPES * N_NODES, MSG), jnp.float32),
        jax.ShapeDtypeStruct((3, N_TYPES, TSTRIDE, MSG), jnp.float32),
    ]
    nb = N_NODES // _ROW_BLK
    return pl.pallas_call(
        _ab_body,
        grid=grid,
        in_specs=[
            pl.BlockSpec((_ROW_BLK, HIDDEN), lambda t, c, i: (i, 0)),
            pl.BlockSpec((1, 2 * HIDDEN, MSG), lambda t, c, i: (t, 0, c)),
            pl.BlockSpec((1, 1, MSG), lambda t, c, i: (t, 0, c)),
        ],
        out_specs=[
            pl.BlockSpec((1, _ROW_BLK, MSG), lambda t, c, i: (c, t * nb + i, 0)),
            pl.BlockSpec((1, 1, _ROW_BLK, MSG), lambda t, c, i: (c, t, i, 0)),
        ],
        out_shape=out_shape,
    )(x, W, b[:, None, :])


def _zero_acc(ref, rows):
    z = jnp.zeros((16,), jnp.float32)

    def body(i, _):
        for j in range(MSG // 16):
            ref[i, pl.ds(j * 16, 16)] = z
        return 0

    lax.fori_loop(0, rows, body, 0)


@functools.partial(
    pl.kernel,
    out_type=[
        jax.ShapeDtypeStruct((OWN_ROWS, MSG), jnp.float32),
        jax.ShapeDtypeStruct((OWN_ROWS, MSG), jnp.float32),
        jax.ShapeDtypeStruct((OWN_ROWS, MSG), jnp.float32),
        jax.ShapeDtypeStruct((OWN_ROWS, MSG), jnp.float32),
    ],
    mesh=plsc.VectorSubcoreMesh(core_axis_name="c", subcore_axis_name="s"),
    compiler_params=pltpu.CompilerParams(needs_layout_passes=False),
    scratch_types=[
        pltpu.VMEM((WL_CAP,), jnp.int32),
        pltpu.VMEM((WL_CAP,), jnp.int32),
        pltpu.VMEM((SCAN_BLK,), jnp.int32),
        pltpu.VMEM((SCAN_BLK,), jnp.int32),
        pltpu.VMEM((SCAN_BLK,), jnp.int32),
        pltpu.VMEM((SCAN_BLK,), jnp.int32),
        pltpu.VMEM((SCAN_BLK,), jnp.int32),
        pltpu.VMEM((SCAN_BLK,), jnp.int32),
        pltpu.VMEM((KB, MSG), jnp.float32),
        pltpu.VMEM((KB, MSG), jnp.float32),
        pltpu.VMEM((KB, MSG), jnp.float32),
        pltpu.VMEM((KB, MSG), jnp.float32),
        pltpu.VMEM((KB, MSG), jnp.float32),
        pltpu.VMEM((KB, MSG), jnp.float32),
        pltpu.VMEM((KB, MSG), jnp.float32),
        pltpu.VMEM((KB, MSG), jnp.float32),
        pltpu.VMEM((OWN, MSG), jnp.float32),
        pltpu.VMEM((OWN, MSG), jnp.float32),
        pltpu.VMEM((OWN,), jnp.float32),
        pltpu.SemaphoreType.DMA,
        pltpu.SemaphoreType.DMA,
        pltpu.SemaphoreType.DMA,
        pltpu.SemaphoreType.DMA,
        pltpu.SemaphoreType.DMA,
        pltpu.SemaphoreType.DMA,
        pltpu.SemaphoreType.DMA,
        pltpu.SemaphoreType.DMA,
    ],
)
def _sc_aggr(a0, b0, a1, b1, a2, b2, sadj, tadj, traw,
             sum_out, mean_out, std_out, max_out,
             wl_s, wl_t, sstg0, sstg1, tstg0, tstg1, rstg0, rstg1,
             ar0, ar1, ar2, ar3, br0, br1, br2, br3,
             acc1, acc2, cnt, semA0, semA1, semA2, semA3,
             semB0, semB1, semB2, semB3):
    c = lax.axis_index("c")
    s = lax.axis_index("s")
    wid = c * 16 + s
    lo = wid * OWN

    iota = lax.iota(jnp.int32, 16)
    zero16i = jnp.zeros((16,), jnp.int32)
    zero16 = jnp.zeros((16,), jnp.float32)
    eps16 = jnp.full((16,), SMALL_NUMBER, jnp.float32)

    def wlz(i, _):
        wl_s[pl.ds(i * 16, 16)] = zero16i
        wl_t[pl.ds(i * 16, 16)] = zero16i
        return 0

    lax.fori_loop(0, WL_CAP // 16, wlz, 0)

    stg = ((sstg0, tstg0, rstg0, semA0, semB0, semA2),
           (sstg1, tstg1, rstg1, semA1, semB1, semA3))

    def scan_issue(bi, par):
        sb, tb, rb, sa, sb_, sc_ = stg[par]
        base = jnp.minimum(bi, N_SCAN - 1) * SCAN_BLK
        pltpu.async_copy(sadj.at[pl.ds(base, SCAN_BLK)], sb, sa)
        pltpu.async_copy(tadj.at[pl.ds(base, SCAN_BLK)], tb, sb_)
        pltpu.async_copy(traw.at[pl.ds(base, SCAN_BLK)], rb, sc_)

    def scan_wait(par):
        sb, tb, rb, sa, sb_, sc_ = stg[par]
        pltpu.make_async_copy(sadj.at[pl.ds(0, SCAN_BLK)], sb, sa).wait()
        pltpu.make_async_copy(tadj.at[pl.ds(0, SCAN_BLK)], tb, sb_).wait()
        pltpu.make_async_copy(traw.at[pl.ds(0, SCAN_BLK)], rb, sc_).wait()

    def scan_body(par, bi, n):
        sb, tb, rb = stg[par][0], stg[par][1], stg[par][2]
        scan_wait(par)

        def grp(g, n):
            sl = pl.ds(g * 16, 16)
            tr = rb[sl]
            m = (tr >= lo) & (tr < lo + OWN)
            nc = jnp.minimum(n, WL_CAP - 128)
            mv = m.astype(jnp.int32)
            pos = plsc.cumsum(mv) + jnp.broadcast_to(nc - 1, (16,))
            plsc.store_scatter(wl_s, [pos], sb[sl], mask=m)
            plsc.store_scatter(wl_t, [pos], tb[sl], mask=m)
            return n + jnp.sum(mv)

        n = plsc.parallel_loop(0, SCAN_BLK // 16, 1, unroll=2, carry=n)(
            lambda g, nn: grp(g, nn))
        scan_issue(bi + 2, par)
        return n

    scan_issue(0, 0)
    scan_issue(1, 1)

    def scan_pair(pi, n):
        n = scan_body(0, pi * 2, n)
        n = scan_body(1, pi * 2 + 1, n)
        return n

    cnt_e = lax.fori_loop(0, N_SCAN // 2, scan_pair, jnp.int32(0))
    scan_wait(0)
    scan_wait(1)

    nblocks = (cnt_e + KB - 1) // KB
    nquad = (nblocks + 4) // 4

    def sweep(atab, btab, update, par_safe, update2=None):
        def issue(b, ar, br, sa, sb_):
            base = b * KB
            pltpu.async_copy(atab.at[wl_s.at[pl.ds(base, KB)]], ar, sa)
            pltpu.async_copy(btab.at[wl_t.at[pl.ds(base, KB)]], br, sb_)

        def wait(ar, br, sa, sb_):
            pltpu.make_async_copy(atab.at[wl_s.at[pl.ds(0, KB)]], ar, sa).wait()
            pltpu.make_async_copy(btab.at[wl_t.at[pl.ds(0, KB)]], br, sb_).wait()

        def proc(b, ar, br, sa, sb_):
            wait(ar, br, sa, sb_)

            def edge_body(e, upd):
                i = b * KB + e
                valid = jnp.broadcast_to(i < cnt_e, (16,))
                tv = plsc.load_gather(wl_t, [jnp.broadcast_to(i, (16,))])
                tloc = jnp.clip((tv & (TSTRIDE - 1)) - lo, 0, OWN - 1)
                upd(ar, br, e, tloc, valid)

            if par_safe:
                @plsc.parallel_loop(0, KB, 1, unroll=2)
                def _(e):
                    edge_body(e, update)
            else:
                def edge(e2, _):
                    edge_body(2 * e2, update)
                    edge_body(2 * e2 + 1, update2)
                    return 0

                lax.fori_loop(0, KB // 2, edge, 0)
            issue(b + 4, ar, br, sa, sb_)

        bufs = ((ar0, br0, semA0, semB0), (ar1, br1, semA1, semB1),
                (ar2, br2, semA2, semB2), (ar3, br3, semA3, semB3))
        for q, (ar, br, sa, sb_) in enumerate(bufs):
            issue(q, ar, br, sa, sb_)

        def quad(qi, _):
            for q, (ar, br, sa, sb_) in enumerate(bufs):
                proc(qi * 4 + q, ar, br, sa, sb_)
            return 0

        lax.fori_loop(0, nquad, quad, 0)
        for ar, br, sa, sb_ in bufs:
            wait(ar, br, sa, sb_)

    def upd_sum(ar, br, e, tloc, valid):
        for j in range(MSG // 16):
            sl = pl.ds(j * 16, 16)
            vals = jnp.maximum(ar[e, sl] + br[e, sl], zero16)
            cols = iota + (j * 16)
            plsc.addupdate_scatter(acc1, [tloc, cols], vals, mask=valid)

    def upd_mean(ar, br, e, tloc, valid):
        upd_sum(ar, br, e, tloc, valid)
        vm = valid & (iota == 0)
        plsc.addupdate_scatter(cnt, [tloc], jnp.ones((16,), jnp.float32),
                               mask=vm)

    def upd_std(ar, br, e, tloc, valid):
        for j in range(MSG // 16):
            sl = pl.ds(j * 16, 16)
            m = jnp.maximum(ar[e, sl] + br[e, sl], zero16)
            cols = iota + (j * 16)
            mu = plsc.load_gather(acc1, [tloc, cols])
            pe = jnp.maximum(m * m - mu * mu, zero16) + eps16
            plsc.addupdate_scatter(acc2, [tloc, cols], pe, mask=valid)

    def _max_into(acc, ar, br, e, tloc, valid):
        news = []
        for j in range(MSG // 16):
            sl = pl.ds(j * 16, 16)
            vals = jnp.maximum(ar[e, sl] + br[e, sl], zero16)
            cols = iota + (j * 16)
            old = plsc.load_gather(acc, [tloc, cols])
            news.append(jnp.maximum(old, vals))
        for j in range(MSG // 16):
            cols = iota + (j * 16)
            plsc.store_scatter(acc, [tloc, cols], news[j], mask=valid)

    def upd_max(ar, br, e, tloc, valid):
        _max_into(acc1, ar, br, e, tloc, valid)

    def upd_max2(ar, br, e, tloc, valid):
        _max_into(acc2, ar, br, e, tloc, valid)

    _zero_acc(acc1, OWN)
    sweep(a0, b0, upd_sum, True)
    pltpu.sync_copy(acc1, sum_out.at[pl.ds(lo, OWN)])

    _zero_acc(acc1, OWN)

    def cz(i, _):
        cnt[pl.ds(i * 16, 16)] = zero16
        return 0

    lax.fori_loop(0, OWN // 16, cz, 0)
    sweep(a1, b1, upd_mean, True)

    def div_row(i, _):
        d = jnp.maximum(plsc.load_gather(cnt, [jnp.broadcast_to(i, (16,))]), 1.0)
        for j in range(MSG // 16):
            sl = pl.ds(j * 16, 16)
            acc1[i, sl] = acc1[i, sl] / d
        return 0

    lax.fori_loop(0, OWN, div_row, 0)
    pltpu.sync_copy(acc1, mean_out.at[pl.ds(lo, OWN)])

    _zero_acc(acc2, OWN)
    sweep(a1, b1, upd_std, True)
    pltpu.sync_copy(acc2, std_out.at[pl.ds(lo, OWN)])

    _zero_acc(acc1, OWN)
    _zero_acc(acc2, OWN)
    sweep(a2, b2, upd_max, False, upd_max2)

    def mmerge(i, _):
        for j in range(MSG // 16):
            sl = pl.ds(j * 16, 16)
            acc1[i, sl] = jnp.maximum(acc1[i, sl], acc2[i, sl])
        return 0

    lax.fori_loop(0, OWN, mmerge, 0)
    pltpu.sync_copy(acc1, max_out.at[pl.ds(lo, OWN)])


def _assemble_body(sum_ref, mean_ref, stp_ref, mx_ref, out_ref):
    std_agg = jnp.sqrt(stp_ref[...])
    out_ref[...] = jnp.concatenate(
        [sum_ref[...], mean_ref[...], std_agg, mx_ref[...]], axis=1)


def _assemble(sum_agg, mean_agg, std_sums, max_agg):
    blk = 1000
    grid = (N_NODES // blk,)
    return pl.pallas_call(
        _assemble_body,
        grid=grid,
        in_specs=[pl.BlockSpec((blk, MSG), lambda i: (i, 0))] * 4,
        out_specs=pl.BlockSpec((blk, 4 * MSG), lambda i: (i, 0)),
        out_shape=jax.ShapeDtypeStruct((N_NODES, 4 * MSG), jnp.float32),
    )(sum_agg, mean_agg, std_sums, max_agg)


def kernel(x, adj_lists, W, b):
    A, B = _compute_ab(x, W, b)

    srcs = adj_lists[:, :, 0]
    tgts = adj_lists[:, :, 1]
    offs = (jnp.arange(N_TYPES, dtype=jnp.int32) * N_NODES)[:, None]
    offs_t = (jnp.arange(N_TYPES, dtype=jnp.int32) * TSTRIDE)[:, None]
    src_adj = (srcs + offs).reshape(-1)
    tgt_adj = (tgts + offs_t).reshape(-1)
    tgt_raw = tgts.reshape(-1)
    pad = EDGES_P - N_EDGES
    src_adj = jnp.concatenate([src_adj, jnp.zeros((pad,), jnp.int32)])
    tgt_adj = jnp.concatenate([tgt_adj, jnp.zeros((pad,), jnp.int32)])
    tgt_raw = jnp.concatenate([tgt_raw, jnp.full((pad,), 1 << 20, jnp.int32)])

    Bf = B.reshape(3, N_TYPES * TSTRIDE, MSG)
    sum_agg, mean_agg, std_sums, max_agg = _sc_aggr(
        A[0], Bf[0], A[1], Bf[1], A[2], Bf[2], src_adj, tgt_adj, tgt_raw)

    return _assemble(sum_agg[:N_NODES], mean_agg[:N_NODES],
                     std_sums[:N_NODES], max_agg[:N_NODES])
